# baseline (speedup 1.0000x reference)
import jax
import jax.numpy as jnp
from jax import lax
from jax.experimental import pallas as pl
from jax.experimental.pallas import tpu as pltpu
from jax.experimental.pallas import tpu_sc as plsc

B, T, E = 32, 2048, 64
V = 1000000
NC, NS, L = 2, 16, 16
NW = NC * NS
TPW = T
CHUNK = 128
NCHUNK = TPW // CHUNK
HROW = CHUNK // 2
NSLOT = 4
OPW = TPW // 2


def _body(idx_hbm, tok_hbm, pos_hbm, out_hbm,
          idx_v, idx2_v, off_v, g_v, pos_v, sin, sout):
    cid = lax.axis_index("c")
    sid = lax.axis_index("s")
    w = sid * NC + cid
    pltpu.sync_copy(idx_hbm.at[pl.ds(w * TPW, TPW)], idx_v)

    def prep(j, carry):
        v = idx_v[pl.ds(j * L, L)]
        idx2_v[pl.ds(j * L, L)] = lax.shift_right_logical(v, 1)
        off_v[pl.ds(j * L, L)] = lax.shift_left(lax.bitwise_and(v, 1), 6)
        return carry

    lax.fori_loop(0, TPW // L, prep, 0)

    def issue_in(c):
        s = c % NSLOT
        g = pltpu.async_copy(
            tok_hbm.at[idx2_v.at[pl.ds(c * CHUNK, CHUNK)]], g_v.at[s], sin[s]
        )
        p = pltpu.async_copy(
            pos_hbm.at[pl.ds(c * HROW, HROW)], pos_v.at[s], sin[s]
        )
        return (g, p)

    def issue_out(c):
        s = c % NSLOT
        return pltpu.async_copy(
            g_v.at[s, pl.ds(0, HROW)],
            out_hbm.at[pl.ds(w * OPW + c * HROW, HROW)],
            sout[s],
        )

    def compute(c):
        s = c % NSLOT
        base = c * CHUNK

        def group(j, carry):
            offs = off_v[pl.ds(base + j * L, L)]
            i0 = j * (L // 2)
            for r in range(L // 2):
                i = i0 + r
                o0 = offs[2 * r]
                o1 = offs[2 * r + 1]
                for k in range(E // L):
                    a = g_v[s, 2 * i, pl.ds(o0 + k * L, L)]
                    p = pos_v[s, i, pl.ds(k * L, L)]
                    g_v[s, i, pl.ds(k * L, L)] = a + p
                for k in range(E // L):
                    a = g_v[s, 2 * i + 1, pl.ds(o1 + k * L, L)]
                    p = pos_v[s, i, pl.ds(E + k * L, L)]
                    g_v[s, i, pl.ds(E + k * L, L)] = a + p
            return carry

        lax.fori_loop(0, CHUNK // L, group, 0)

    d_in = {0: issue_in(0), 1: issue_in(1)}
    d_out = {}
    for c in range(NCHUNK):
        if c + 2 < NCHUNK:
            if c - 2 >= 0:
                d_out.pop(c - 2).wait()
            d_in[c + 2] = issue_in(c + 2)
        g, p = d_in.pop(c)
        g.wait()
        p.wait()
        compute(c)
        d_out[c] = issue_out(c)
    d_out.pop(NCHUNK - 2).wait()
    d_out.pop(NCHUNK - 1).wait()


@jax.jit
def _emb(idx_flat, tok2, pos2):
    mesh = plsc.VectorSubcoreMesh(
        core_axis_name="c", subcore_axis_name="s", num_cores=NC, num_subcores=NS
    )
    f = pl.kernel(
        _body,
        out_type=jax.ShapeDtypeStruct((B * T // 2, 128), jnp.float32),
        mesh=mesh,
        scratch_types=[
            pltpu.VMEM((TPW,), jnp.int32),
            pltpu.VMEM((TPW,), jnp.int32),
            pltpu.VMEM((TPW,), jnp.int32),
            pltpu.VMEM((NSLOT, CHUNK, 128), jnp.float32),
            pltpu.VMEM((NSLOT, HROW, 128), jnp.float32),
            [pltpu.SemaphoreType.DMA] * NSLOT,
            [pltpu.SemaphoreType.DMA] * NSLOT,
        ],
    )
    return f(idx_flat, tok2, pos2)


def kernel(idx, token_table, pos_table):
    out = _emb(
        idx.reshape(-1).astype(jnp.int32),
        token_table.reshape(V // 2, 128),
        pos_table.reshape(T // 2, 128),
    )
    return out.reshape(B, T, E)
